# baseline (speedup 1.0000x reference)
import functools

import jax
import jax.numpy as jnp
from jax.experimental import pallas as pl

N = 100000
D = 128
BLOCK = 2000


def _deletion_kernel(x_ref, m_ref, w_ref, o_ref):
    xb = x_ref[...]
    t = jnp.dot(xb, w_ref[...], preferred_element_type=jnp.float32)
    o_ref[...] = jnp.where(m_ref[...], t, xb)


@jax.jit
def _run(x, mask2d, w):
    grid = (N // BLOCK,)
    return pl.pallas_call(
        _deletion_kernel,
        grid=grid,
        in_specs=[
            pl.BlockSpec((BLOCK, D), lambda i: (i, 0)),
            pl.BlockSpec((BLOCK, 1), lambda i: (i, 0)),
            pl.BlockSpec((D, D), lambda i: (0, 0)),
        ],
        out_specs=pl.BlockSpec((BLOCK, D), lambda i: (i, 0)),
        out_shape=jax.ShapeDtypeStruct((N, D), jnp.float32),
    )(x, mask2d, w)


def kernel(x, mask, deletion_weight):
    return _run(x, mask.reshape(N, 1), deletion_weight)


# speedup vs baseline: 1.7187x; 1.7187x over previous
import jax
import jax.numpy as jnp
from jax.experimental import pallas as pl
from jax.experimental.pallas import tpu as pltpu

N = 100000
D = 128
BLOCK = 2000
GRID = N // BLOCK


def _deletion_kernel(x_ref, m_ref, v_ref, ones_ref, o_ref):
    xb = x_ref[...]
    t = jnp.dot(xb, v_ref[...], preferred_element_type=jnp.float32)
    m_row = m_ref[0]
    mcol = jax.lax.dot_general(
        m_row, ones_ref[...],
        dimension_numbers=(((0,), (0,)), ((), ())),
        preferred_element_type=jnp.float32,
    )
    o_ref[...] = xb + mcol * t


@jax.jit
def _run(x, m3d, v, ones_row):
    return pl.pallas_call(
        _deletion_kernel,
        grid=(GRID,),
        in_specs=[
            pl.BlockSpec((BLOCK, D), lambda i: (i, 0)),
            pl.BlockSpec((1, 1, BLOCK), lambda i: (i, 0, 0)),
            pl.BlockSpec((D, D), lambda i: (0, 0)),
            pl.BlockSpec((1, D), lambda i: (0, 0)),
        ],
        out_specs=pl.BlockSpec((BLOCK, D), lambda i: (i, 0)),
        out_shape=jax.ShapeDtypeStruct((N, D), jnp.float32),
        compiler_params=pltpu.CompilerParams(
            dimension_semantics=("parallel",),
        ),
    )(x, m3d, v, ones_row)


def kernel(x, mask, deletion_weight):
    v = deletion_weight - jnp.eye(D, dtype=jnp.float32)
    m3d = mask.astype(jnp.float32).reshape(GRID, 1, BLOCK)
    ones_row = jnp.ones((1, D), dtype=jnp.float32)
    return _run(x, m3d, v, ones_row)


# speedup vs baseline: 2.7775x; 1.6160x over previous
import jax
import jax.numpy as jnp
from jax.experimental import pallas as pl
from jax.experimental.pallas import tpu as pltpu

N = 100000
D = 128
BLOCK = 10000
GRID = N // BLOCK


def _deletion_kernel(x_ref, m_ref, v_ref, ones_ref, o_ref):
    xb = x_ref[...]
    t = jnp.dot(xb, v_ref[...], preferred_element_type=jnp.float32)
    m_row = m_ref[0]
    mcol = jax.lax.dot_general(
        m_row, ones_ref[...],
        dimension_numbers=(((0,), (0,)), ((), ())),
        preferred_element_type=jnp.float32,
    )
    o_ref[...] = xb + mcol * t


@jax.jit
def _run(x, m3d, v, ones_row):
    return pl.pallas_call(
        _deletion_kernel,
        grid=(GRID,),
        in_specs=[
            pl.BlockSpec((BLOCK, D), lambda i: (i, 0)),
            pl.BlockSpec((1, 1, BLOCK), lambda i: (i, 0, 0)),
            pl.BlockSpec((D, D), lambda i: (0, 0)),
            pl.BlockSpec((1, D), lambda i: (0, 0)),
        ],
        out_specs=pl.BlockSpec((BLOCK, D), lambda i: (i, 0)),
        out_shape=jax.ShapeDtypeStruct((N, D), jnp.float32),
        compiler_params=pltpu.CompilerParams(
            dimension_semantics=("parallel",),
        ),
    )(x, m3d, v, ones_row)


def kernel(x, mask, deletion_weight):
    v = deletion_weight - jnp.eye(D, dtype=jnp.float32)
    m3d = mask.astype(jnp.float32).reshape(GRID, 1, BLOCK)
    ones_row = jnp.ones((1, D), dtype=jnp.float32)
    return _run(x, m3d, v, ones_row)


# speedup vs baseline: 2.7988x; 1.0077x over previous
import jax
import jax.numpy as jnp
from jax.experimental import pallas as pl
from jax.experimental.pallas import tpu as pltpu

N = 100000
D = 128
BLOCK = 20000
GRID = N // BLOCK


def _deletion_kernel(x_ref, m_ref, v_ref, ones_ref, o_ref):
    xb = x_ref[...]
    t = jnp.dot(xb, v_ref[...], preferred_element_type=jnp.float32)
    m_row = m_ref[0]
    mcol = jax.lax.dot_general(
        m_row, ones_ref[...],
        dimension_numbers=(((0,), (0,)), ((), ())),
        preferred_element_type=jnp.float32,
    )
    o_ref[...] = xb + mcol * t


@jax.jit
def _run(x, m3d, v, ones_row):
    return pl.pallas_call(
        _deletion_kernel,
        grid=(GRID,),
        in_specs=[
            pl.BlockSpec((BLOCK, D), lambda i: (i, 0)),
            pl.BlockSpec((1, 1, BLOCK), lambda i: (i, 0, 0)),
            pl.BlockSpec((D, D), lambda i: (0, 0)),
            pl.BlockSpec((1, D), lambda i: (0, 0)),
        ],
        out_specs=pl.BlockSpec((BLOCK, D), lambda i: (i, 0)),
        out_shape=jax.ShapeDtypeStruct((N, D), jnp.float32),
        compiler_params=pltpu.CompilerParams(
            dimension_semantics=("parallel",),
        ),
    )(x, m3d, v, ones_row)


def kernel(x, mask, deletion_weight):
    v = deletion_weight - jnp.eye(D, dtype=jnp.float32)
    m3d = mask.astype(jnp.float32).reshape(GRID, 1, BLOCK)
    ones_row = jnp.ones((1, D), dtype=jnp.float32)
    return _run(x, m3d, v, ones_row)
